# baseline (speedup 1.0000x reference)
import functools

import jax
import jax.numpy as jnp
from jax import lax
from jax.experimental import pallas as pl
from jax.experimental.pallas import tpu as pltpu
from jax.experimental.pallas import tpu_sc as plsc

N = 10000
E = 320000
D = 128

NC = 2
NS = 16
L = 16

CH = 80
EPT = E // NS
NCH = EPT // CH
NCHP = 256
NIT = 16
CPI = NCHP // NIT
RPT = 624
REM = N - NS * RPT
GBYTES = CH * D * 4


def _matmul_body(x_ref, w_ref, y_ref):
    y_ref[0] = jnp.dot(x_ref[...], w_ref[0], preferred_element_type=jnp.float32)


def _projections(x, W2):
    br = N // 25
    return pl.pallas_call(
        _matmul_body,
        grid=(2, 25),
        in_specs=[
            pl.BlockSpec((br, D), lambda g, i: (i, 0)),
            pl.BlockSpec((1, D, D), lambda g, i: (g, 0, 0)),
        ],
        out_specs=pl.BlockSpec((1, br, D), lambda g, i: (g, i, 0)),
        out_shape=jax.ShapeDtypeStruct((2, N, D), jnp.float32),
    )(x, W2)


_BCAST_DN = lax.GatherDimensionNumbers(
    offset_dims=(), collapsed_slice_dims=(0,), start_index_map=(0,))


def _sc_body(y_hbm, e_hbm, v_hbm, part_hbm, ebuf, vbuf, rows2, acc_sh, gsem, ssem):
    c = lax.axis_index("c")
    s = lax.axis_index("s")

    def zrow(r, carry):
        for k in range(D // L):
            rows2[0, r, pl.ds(k * L, L)] = jnp.zeros((L,), jnp.float32)
        return carry
    lax.fori_loop(0, CH, zrow, 0)
    for i in range(RPT // CH):
        pltpu.sync_copy(rows2.at[0], acc_sh.at[pl.ds(s * RPT + i * CH, CH)])
    zr = RPT - (RPT // CH) * CH
    pltpu.sync_copy(rows2.at[0, pl.ds(0, zr)],
                    acc_sh.at[pl.ds(s * RPT + RPT - zr, zr)])

    @pl.when(s == 0)
    def _zero_rem():
        pltpu.sync_copy(rows2.at[0, pl.ds(0, REM)],
                        acc_sh.at[pl.ds(NS * RPT, REM)])

    plsc.subcore_barrier()

    def g_start(j, b):
        pltpu.async_copy(y_hbm.at[ebuf.at[0, j]], rows2.at[b], gsem.at[b])

    def g_wait(j, b):
        pltpu.make_async_copy(
            y_hbm.at[ebuf.at[0, j]], rows2.at[b], gsem.at[b]).wait()

    def s_start(j, b):
        pltpu.async_copy(rows2.at[b], acc_sh.at[ebuf.at[1, j]], ssem.at[b],
                         add=True)

    def s_wait(j, b):
        pltpu.make_async_copy(
            rows2.at[b], acc_sh.at[ebuf.at[1, j]], ssem.at[b]).wait()

    zlane = lax.iota(jnp.int32, L) * 0

    def compute(j, b):
        def group(g, carry):
            vals16 = vbuf[j, pl.ds(g * L, L)]
            for e in range(L):
                vb = lax.gather(
                    vals16, (zlane + e).reshape(L, 1), _BCAST_DN,
                    slice_sizes=(1,),
                    mode=lax.GatherScatterMode.PROMISE_IN_BOUNDS)
                row = g * L + e
                for k in range(D // L):
                    rows2[b, row, pl.ds(k * L, L)] = (
                        rows2[b, row, pl.ds(k * L, L)] * vb)
            return carry
        lax.fori_loop(0, CH // L, group, 0)

    def it_body(it, carry):
        pltpu.sync_copy(e_hbm.at[c, s, it], ebuf)
        pltpu.sync_copy(v_hbm.at[c, s, it], vbuf)
        g_start(0, 0)

        def slot(j, carry2):
            b = jnp.bitwise_and(j, 1)
            nb = 1 - b

            @pl.when(j < CPI - 1)
            def _prefetch():
                @pl.when(j > 0)
                def _drain_prev():
                    s_wait(j - 1, nb)
                g_start(j + 1, nb)

            g_wait(j, b)
            compute(j, b)
            s_start(j, b)
            return carry2
        lax.fori_loop(0, CPI, slot, 0)
        s_wait(CPI - 2, 0)
        s_wait(CPI - 1, 1)
        return carry
    lax.fori_loop(0, NIT, it_body, 0)

    plsc.subcore_barrier()
    pltpu.sync_copy(acc_sh.at[pl.ds(s * RPT, RPT)],
                    part_hbm.at[c, pl.ds(s * RPT, RPT)])

    @pl.when(s == 0)
    def _copy_rem():
        pltpu.sync_copy(acc_sh.at[pl.ds(NS * RPT, REM)],
                        part_hbm.at[c, pl.ds(NS * RPT, REM)])


def _sparse_partials(y2, edges, vals):
    mesh = plsc.VectorSubcoreMesh(
        core_axis_name="c", subcore_axis_name="s", num_cores=NC, num_subcores=NS)
    fn = pl.kernel(
        _sc_body,
        out_type=jax.ShapeDtypeStruct((NC, N, D), jnp.float32),
        mesh=mesh,
        scratch_types=[
            pltpu.VMEM((2, CPI, CH), jnp.int32),
            pltpu.VMEM((CPI, CH), jnp.float32),
            pltpu.VMEM((2, CH, D), jnp.float32),
            pltpu.VMEM_SHARED((N, D), jnp.float32),
            pltpu.SemaphoreType.DMA((2,)),
            pltpu.SemaphoreType.DMA((2,)),
        ],
    )
    return fn(y2.reshape(2 * N, D), edges, vals)


def _add_body(p_ref, o_ref):
    o_ref[...] = p_ref[0] + p_ref[1]


def _sum_partials(part):
    br = N // 25
    return pl.pallas_call(
        _add_body,
        grid=(25,),
        in_specs=[pl.BlockSpec((2, br, D), lambda i: (0, i, 0))],
        out_specs=pl.BlockSpec((br, D), lambda i: (i, 0)),
        out_shape=jax.ShapeDtypeStruct((N, D), jnp.float32),
    )(part)


def _pack_edges(lower_index, lower_values, upper_index, upper_values):
    def one_idx(idx, base):
        arr = jnp.stack([idx[1] + base, idx[0]])
        arr = arr.reshape(2, NS, NCH, CH)
        arr = jnp.pad(arr, ((0, 0), (0, 0), (0, NCHP - NCH), (0, 0)))
        arr = arr.reshape(2, NS, NIT, CPI, CH)
        return arr.transpose(1, 2, 0, 3, 4)

    def one_val(val):
        arr = val.reshape(NS, NCH, CH)
        arr = jnp.pad(arr, ((0, 0), (0, NCHP - NCH), (0, 0)))
        return arr.reshape(NS, NIT, CPI, CH)

    edges = jnp.stack([one_idx(lower_index, 0), one_idx(upper_index, N)])
    vals = jnp.stack([one_val(lower_values), one_val(upper_values)])
    return edges, vals


def kernel(x, lower_index, lower_values, upper_index, upper_values, W_irr, W_sol):
    W2 = jnp.stack([W_irr, W_sol])
    y2 = _projections(x, W2)
    edges, vals = _pack_edges(lower_index, lower_values, upper_index, upper_values)
    part = _sparse_partials(y2, edges, vals)
    return _sum_partials(part)
